# initial kernel scaffold (unmeasured)
import jax
import jax.numpy as jnp
from jax import lax
from jax.experimental import pallas as pl
from jax.experimental.pallas import tpu as pltpu

N_RING = 4


def kernel(O, Wo):
    Bv, Sg, Hl, D = O.shape
    K = Hl * D
    n = Wo.shape[1]
    s_per = Sg // N_RING
    rows = Bv * s_per

    o_flat = O.reshape(Bv * Sg, K).astype(jnp.bfloat16)
    w = Wo.astype(jnp.bfloat16)

    def body(o_ref, w_ref, out_ref, comm_ref, send_sems, recv_sems):
        my_x = lax.axis_index("x")
        my_y = lax.axis_index("y")
        my_z = lax.axis_index("z")
        left = jnp.mod(my_y + N_RING - 1, N_RING)
        right = jnp.mod(my_y + 1, N_RING)

        barrier_sem = pltpu.get_barrier_semaphore()
        for nbr in (left, right):
            pl.semaphore_signal(
                barrier_sem, inc=1,
                device_id=(my_x, nbr, my_z),
                device_id_type=pl.DeviceIdType.MESH,
            )
        pl.semaphore_wait(barrier_sem, 2)

        def partials(c):
            r = c * s_per
            a0 = o_ref[pl.ds(r, s_per), :]
            a1 = o_ref[pl.ds(Sg + r, s_per), :]
            p0 = jnp.dot(a0, w_ref[:, :], preferred_element_type=jnp.float32)
            p1 = jnp.dot(a1, w_ref[:, :], preferred_element_type=jnp.float32)
            return p0, p1

        c0 = jnp.mod(my_y + N_RING - 1, N_RING)
        p0, p1 = partials(c0)
        comm_ref[0, 0:s_per, :] = p0.astype(jnp.bfloat16)
        comm_ref[0, s_per:rows, :] = p1.astype(jnp.bfloat16)

        for s in range(N_RING - 1):
            rdma = pltpu.make_async_remote_copy(
                src_ref=comm_ref.at[s],
                dst_ref=comm_ref.at[s + 1],
                send_sem=send_sems.at[s],
                recv_sem=recv_sems.at[s],
                device_id=(my_x, right, my_z),
                device_id_type=pl.DeviceIdType.MESH,
            )
            rdma.start()
            c = jnp.mod(my_y + 2 * N_RING - 2 - s, N_RING)
            q0, q1 = partials(c)
            rdma.wait()
            if s < N_RING - 2:
                acc0 = comm_ref[s + 1, 0:s_per, :].astype(jnp.float32) + q0
                acc1 = comm_ref[s + 1, s_per:rows, :].astype(jnp.float32) + q1
                comm_ref[s + 1, 0:s_per, :] = acc0.astype(jnp.bfloat16)
                comm_ref[s + 1, s_per:rows, :] = acc1.astype(jnp.bfloat16)
            else:
                out_ref[0:s_per, :] = (
                    comm_ref[s + 1, 0:s_per, :].astype(jnp.float32) + q0
                )
                out_ref[s_per:rows, :] = (
                    comm_ref[s + 1, s_per:rows, :].astype(jnp.float32) + q1
                )

    out_flat = pl.pallas_call(
        body,
        out_shape=jax.ShapeDtypeStruct((rows, n), jnp.float32),
        in_specs=[
            pl.BlockSpec(memory_space=pltpu.VMEM),
            pl.BlockSpec(memory_space=pltpu.VMEM),
        ],
        out_specs=pl.BlockSpec(memory_space=pltpu.VMEM),
        scratch_shapes=[
            pltpu.VMEM((N_RING, rows, n), jnp.bfloat16),
            pltpu.SemaphoreType.DMA((N_RING - 1,)),
            pltpu.SemaphoreType.DMA((N_RING - 1,)),
        ],
        compiler_params=pltpu.CompilerParams(collective_id=0),
    )(o_flat, w)
    return out_flat.reshape(Bv, s_per, n)


# baseline (device time: 382254 ns/iter reference)
import jax
import jax.numpy as jnp
from jax import lax
from jax.experimental import pallas as pl
from jax.experimental.pallas import tpu as pltpu

N_RING = 4


def kernel(O, Wo):
    Bv, Sg, Hl, D = O.shape
    K = Hl * D
    n = Wo.shape[1]
    s_per = Sg // N_RING
    rows = Bv * s_per
    nt = n // 2

    o_flat = O.reshape(Bv * Sg, K).astype(jnp.bfloat16)
    w = Wo.astype(jnp.bfloat16)

    def body(o_ref, w_ref, out_ref, comm_ref, send_sems, recv_sems):
        my_x = lax.axis_index("x")
        my_y = lax.axis_index("y")
        my_z = lax.axis_index("z")
        left = jnp.mod(my_y + N_RING - 1, N_RING)
        right = jnp.mod(my_y + 1, N_RING)

        barrier_sem = pltpu.get_barrier_semaphore()
        for nbr in (left, right):
            pl.semaphore_signal(
                barrier_sem, inc=1,
                device_id=(my_x, nbr, my_z),
                device_id_type=pl.DeviceIdType.MESH,
            )
        pl.semaphore_wait(barrier_sem, 2)

        def partial_into(c, dst_ref, add):
            for b in range(Bv):
                a = o_ref[pl.ds(b * Sg + c * s_per, s_per), :]
                for h in range(2):
                    p = jnp.dot(
                        a, w_ref[:, h * nt:(h + 1) * nt],
                        preferred_element_type=jnp.float32,
                    )
                    if add:
                        p = p + dst_ref[
                            b * s_per:(b + 1) * s_per, h * nt:(h + 1) * nt
                        ].astype(jnp.float32)
                    dst_ref[
                        b * s_per:(b + 1) * s_per, h * nt:(h + 1) * nt
                    ] = p.astype(jnp.bfloat16)

        c0 = jnp.mod(my_y + N_RING - 1, N_RING)
        partial_into(c0, comm_ref.at[0], add=False)

        for s in range(N_RING - 1):
            last = s == N_RING - 2
            rdma = pltpu.make_async_remote_copy(
                src_ref=comm_ref.at[s],
                dst_ref=out_ref if last else comm_ref.at[s + 1],
                send_sem=send_sems.at[s],
                recv_sem=recv_sems.at[s],
                device_id=(my_x, right, my_z),
                device_id_type=pl.DeviceIdType.MESH,
            )
            rdma.start()
            rdma.wait()
            c = jnp.mod(my_y + 2 * N_RING - 2 - s, N_RING)
            partial_into(c, out_ref if last else comm_ref.at[s + 1], add=True)

    out_flat = pl.pallas_call(
        body,
        out_shape=jax.ShapeDtypeStruct((rows, n), jnp.bfloat16),
        in_specs=[
            pl.BlockSpec(memory_space=pltpu.VMEM),
            pl.BlockSpec(memory_space=pltpu.VMEM),
        ],
        out_specs=pl.BlockSpec(memory_space=pltpu.VMEM),
        scratch_shapes=[
            pltpu.VMEM((N_RING - 1, rows, n), jnp.bfloat16),
            pltpu.SemaphoreType.DMA((N_RING - 1,)),
            pltpu.SemaphoreType.DMA((N_RING - 1,)),
        ],
        compiler_params=pltpu.CompilerParams(
            collective_id=0,
            vmem_limit_bytes=63 * 1024 * 1024,
        ),
    )(o_flat, w)
    return out_flat.astype(jnp.float32).reshape(Bv, s_per, n)


# device time: 362158 ns/iter; 1.0555x vs baseline; 1.0555x over previous
import jax
import jax.numpy as jnp
from jax import lax
from jax.experimental import pallas as pl
from jax.experimental.pallas import tpu as pltpu

N_RING = 4


def kernel(O, Wo):
    Bv, Sg, Hl, D = O.shape
    K = Hl * D
    n = Wo.shape[1]
    s_per = Sg // N_RING
    rows = Bv * s_per
    nt = n // 4

    o_flat = O.reshape(Bv * Sg, K).astype(jnp.bfloat16)
    w = Wo.astype(jnp.bfloat16)

    def body(o_ref, w_ref, out_ref, comm_ref, p_ref, send_sems, recv_sems):
        my_x = lax.axis_index("x")
        my_y = lax.axis_index("y")
        my_z = lax.axis_index("z")
        left = jnp.mod(my_y + N_RING - 1, N_RING)
        right = jnp.mod(my_y + 1, N_RING)

        barrier_sem = pltpu.get_barrier_semaphore()
        for nbr in (left, right):
            pl.semaphore_signal(
                barrier_sem, inc=1,
                device_id=(my_x, nbr, my_z),
                device_id_type=pl.DeviceIdType.MESH,
            )
        pl.semaphore_wait(barrier_sem, 2)

        def partial_into(c, dst_ref):
            for b in range(Bv):
                a = o_ref[pl.ds(b * Sg + c * s_per, s_per), :]
                for h in range(4):
                    p = jnp.dot(
                        a, w_ref[:, h * nt:(h + 1) * nt],
                        preferred_element_type=jnp.float32,
                    )
                    dst_ref[
                        b * s_per:(b + 1) * s_per, h * nt:(h + 1) * nt
                    ] = p.astype(jnp.bfloat16)

        def add_staged(dst_ref):
            for b in range(Bv):
                for h in range(4):
                    r0, r1 = b * s_per, (b + 1) * s_per
                    c0_, c1_ = h * nt, (h + 1) * nt
                    dst_ref[r0:r1, c0_:c1_] = (
                        dst_ref[r0:r1, c0_:c1_].astype(jnp.float32)
                        + p_ref[r0:r1, c0_:c1_].astype(jnp.float32)
                    ).astype(jnp.bfloat16)

        c0 = jnp.mod(my_y + N_RING - 1, N_RING)
        partial_into(c0, comm_ref.at[0])

        for s in range(N_RING - 1):
            last = s == N_RING - 2
            rdma = pltpu.make_async_remote_copy(
                src_ref=comm_ref.at[s],
                dst_ref=out_ref if last else comm_ref.at[s + 1],
                send_sem=send_sems.at[s],
                recv_sem=recv_sems.at[s],
                device_id=(my_x, right, my_z),
                device_id_type=pl.DeviceIdType.MESH,
            )
            rdma.start()
            c = jnp.mod(my_y + 2 * N_RING - 2 - s, N_RING)
            partial_into(c, p_ref)
            rdma.wait()
            add_staged(out_ref if last else comm_ref.at[s + 1])

    out_flat = pl.pallas_call(
        body,
        out_shape=jax.ShapeDtypeStruct((rows, n), jnp.bfloat16),
        in_specs=[
            pl.BlockSpec(memory_space=pltpu.VMEM),
            pl.BlockSpec(memory_space=pltpu.VMEM),
        ],
        out_specs=pl.BlockSpec(memory_space=pltpu.VMEM),
        scratch_shapes=[
            pltpu.VMEM((N_RING - 1, rows, n), jnp.bfloat16),
            pltpu.VMEM((rows, n), jnp.bfloat16),
            pltpu.SemaphoreType.DMA((N_RING - 1,)),
            pltpu.SemaphoreType.DMA((N_RING - 1,)),
        ],
        compiler_params=pltpu.CompilerParams(
            collective_id=0,
            vmem_limit_bytes=63 * 1024 * 1024,
        ),
    )(o_flat, w)
    return out_flat.astype(jnp.float32).reshape(Bv, s_per, n)


# device time: 346788 ns/iter; 1.1023x vs baseline; 1.0443x over previous
import jax
import jax.numpy as jnp
from jax import lax
from jax.experimental import pallas as pl
from jax.experimental.pallas import tpu as pltpu

N_RING = 4


def kernel(O, Wo):
    Bv, Sg, Hl, D = O.shape
    K = Hl * D
    n = Wo.shape[1]
    s_per = Sg // N_RING
    nt = n // 4

    o_flat = O.reshape(Bv * Sg, K).astype(jnp.bfloat16)
    w = Wo.astype(jnp.bfloat16)

    def body(o_ref, w_ref, out_ref, comm_ref, p_ref, send_sems, recv_sems):
        my_x = lax.axis_index("x")
        my_y = lax.axis_index("y")
        my_z = lax.axis_index("z")
        left = jnp.mod(my_y + N_RING - 1, N_RING)
        right = jnp.mod(my_y + 1, N_RING)

        barrier_sem = pltpu.get_barrier_semaphore()
        for nbr in (left, right):
            pl.semaphore_signal(
                barrier_sem, inc=1,
                device_id=(my_x, nbr, my_z),
                device_id_type=pl.DeviceIdType.MESH,
            )
        pl.semaphore_wait(barrier_sem, 2)

        def dot_batch(c, b, dst2):
            a = o_ref[pl.ds(b * Sg + c * s_per, s_per), :]
            for h in range(4):
                p = jnp.dot(
                    a, w_ref[:, h * nt:(h + 1) * nt],
                    preferred_element_type=jnp.float32,
                )
                dst2[:, h * nt:(h + 1) * nt] = p.astype(jnp.bfloat16)

        def add_batch(b, dst2):
            for h in range(4):
                dst2[:, h * nt:(h + 1) * nt] = (
                    dst2[:, h * nt:(h + 1) * nt].astype(jnp.float32)
                    + p_ref[b, :, h * nt:(h + 1) * nt].astype(jnp.float32)
                ).astype(jnp.bfloat16)

        def mk(s, b):
            last = s == N_RING - 2
            return pltpu.make_async_remote_copy(
                src_ref=comm_ref.at[s, b],
                dst_ref=out_ref.at[b] if last else comm_ref.at[s + 1, b],
                send_sem=send_sems.at[s, b],
                recv_sem=recv_sems.at[s, b],
                device_id=(my_x, right, my_z),
                device_id_type=pl.DeviceIdType.MESH,
            )

        c_first = jnp.mod(my_y + N_RING - 1, N_RING)
        for b in range(Bv):
            dot_batch(c_first, b, comm_ref.at[0, b])
            mk(0, b).start()

        for s in range(N_RING - 1):
            last = s == N_RING - 2
            c = jnp.mod(my_y + 2 * N_RING - 2 - s, N_RING)
            for b in range(Bv):
                dot_batch(c, b, p_ref.at[b])
                mk(s, b).wait_recv()
                add_batch(b, out_ref.at[b] if last else comm_ref.at[s + 1, b])
                if not last:
                    mk(s + 1, b).start()

        for s in range(N_RING - 1):
            for b in range(Bv):
                mk(s, b).wait_send()

    out = pl.pallas_call(
        body,
        out_shape=jax.ShapeDtypeStruct((Bv, s_per, n), jnp.bfloat16),
        in_specs=[
            pl.BlockSpec(memory_space=pltpu.VMEM),
            pl.BlockSpec(memory_space=pltpu.VMEM),
        ],
        out_specs=pl.BlockSpec(memory_space=pltpu.VMEM),
        scratch_shapes=[
            pltpu.VMEM((N_RING - 1, Bv, s_per, n), jnp.bfloat16),
            pltpu.VMEM((Bv, s_per, n), jnp.bfloat16),
            pltpu.SemaphoreType.DMA((N_RING - 1, Bv)),
            pltpu.SemaphoreType.DMA((N_RING - 1, Bv)),
        ],
        compiler_params=pltpu.CompilerParams(
            collective_id=0,
            vmem_limit_bytes=63 * 1024 * 1024,
        ),
    )(o_flat, w)
    return out.astype(jnp.float32)
